# baseline (speedup 1.0000x reference)
import functools

import jax
import jax.numpy as jnp
from jax import lax
from jax.experimental import pallas as pl
from jax.experimental.pallas import tpu as pltpu

B = 32
H = 32
KVH = 4
D = 128
CACHE = 131072
BLOCK = 16
NSLOTS = 256
L_GATHER = NSLOTS * BLOCK

RB = 24
BPC = 64
CR = BPC * RB
NCH = NSLOTS // BPC
T = B * NCH
BIGPOS = 2 ** 30
NBUF = 3

ROWF = KVH * D
SCALE = 1.0 / (D ** 0.5)
NEG = -1e30


def _rope(x, cosv, sinv):
    half = jnp.concatenate([x[..., 64:], x[..., :64]], axis=-1)
    coef = jnp.where(
        lax.broadcasted_iota(jnp.int32, x.shape, x.ndim - 1) < 64, -1.0, 1.0)
    return x * cosv + half * coef * sinv


GRP = 8


def _body(fs_ref, il_ref, ss_ref, hit_ref,
          q_ref, kall_ref, vall_ref, idx_ref, pos_ref, cs_ref, kc_ref,
          vc_ref,
          out_ref,
          kbuf, vbuf, acc_ref, m_ref, l_ref, ksem, vsem):
    t = pl.program_id(0)
    b = t // NCH
    c = t % NCH

    def nblocks(bb):
        nv = jnp.maximum(il_ref[bb] - 1, 0)
        return (nv + BLOCK - 1) // BLOCK

    def issue(tt, sl):
        bb = tt // NCH
        cc = tt % NCH
        nb = jnp.clip(nblocks(bb) - cc * BPC, 0, BPC)
        for g in range(BPC // GRP):
            @pl.when(nb > g * GRP)
            def _():
                for j in range(GRP):
                    i = g * GRP + j
                    i_eff = jnp.minimum(i, nb - 1)
                    row = fs_ref[bb, cc * BPC + i_eff]
                    row8 = pl.multiple_of((row // 8) * 8, 8)
                    pltpu.make_async_copy(
                        kc_ref.at[pl.ds(row8, RB)],
                        kbuf.at[sl, pl.ds(i * RB, RB)],
                        ksem.at[sl]).start()
                    pltpu.make_async_copy(
                        vc_ref.at[pl.ds(row8, RB)],
                        vbuf.at[sl, pl.ds(i * RB, RB)],
                        vsem.at[sl]).start()

    @pl.when(t == 0)
    def _():
        kbuf[...] = jnp.zeros((NBUF, CR, ROWF), jnp.float32)
        vbuf[...] = jnp.zeros((NBUF, CR, ROWF), jnp.float32)
        issue(0, 0)
        issue(1, 1)

    @pl.when(t + 2 < T)
    def _():
        issue(t + 2, (t + 2) % NBUF)

    slot = t % NBUF
    nb_cur = jnp.clip(nblocks(b) - c * BPC, 0, BPC)
    nb_iss = ((nb_cur + GRP - 1) // GRP) * GRP

    @pl.when(nb_iss == BPC)
    def _():
        pltpu.make_async_copy(
            kc_ref.at[pl.ds(0, CR)], kbuf.at[slot], ksem.at[slot]).wait()
        pltpu.make_async_copy(
            vc_ref.at[pl.ds(0, CR)], vbuf.at[slot], vsem.at[slot]).wait()

    @pl.when(jnp.logical_and(nb_iss > 0, nb_iss < BPC))
    def _():
        def wait_one(i, _):
            pltpu.make_async_copy(
                kc_ref.at[pl.ds(0, GRP * RB)],
                kbuf.at[slot, pl.ds(i * GRP * RB, GRP * RB)],
                ksem.at[slot]).wait()
            pltpu.make_async_copy(
                vc_ref.at[pl.ds(0, GRP * RB)],
                vbuf.at[slot, pl.ds(i * GRP * RB, GRP * RB)],
                vsem.at[slot]).wait()
            return 0

        lax.fori_loop(0, nb_iss // GRP, wait_one, 0)

    @pl.when(c == 0)
    def _():
        m_ref[...] = jnp.full((H, 128), NEG, jnp.float32)
        l_ref[...] = jnp.zeros((H, 128), jnp.float32)
        acc_ref[...] = jnp.zeros((H, D), jnp.float32)

    cosv = cs_ref[0, 0]
    sinv = cs_ref[0, 1]

    nvalid = jnp.maximum(il_ref[b] - 1, 0)

    @pl.when(hit_ref[t] > 0)
    def _():
        ka = kall_ref[...]
        kr3 = _rope(ka, cosv, sinv)
        kr_flat = jnp.concatenate(
            [kr3[:, h, :] for h in range(KVH)], axis=1)
        v_flat = jnp.concatenate(
            [vall_ref[:, h, :] for h in range(KVH)], axis=1)
        idxblk2 = idx_ref[0]
        wr = jnp.full((1, CR), -1, jnp.int32)
        for bp in range(B):
            hit = jnp.logical_and(idxblk2 == ss_ref[bp], bp < b)
            wr = jnp.where(hit, bp, wr)
        onehot = (lax.broadcasted_iota(jnp.int32, (B, CR), 0) == wr
                  ).astype(jnp.float32)
        kover = lax.dot_general(
            onehot, kr_flat, (((0,), (0,)), ((), ())),
            preferred_element_type=jnp.float32,
            precision=lax.Precision.DEFAULT)
        vover = lax.dot_general(
            onehot, v_flat, (((0,), (0,)), ((), ())),
            preferred_element_type=jnp.float32,
            precision=lax.Precision.DEFAULT)
        flag = lax.dot_general(
            onehot, jnp.ones((B, 128), jnp.float32),
            (((0,), (0,)), ((), ())),
            preferred_element_type=jnp.float32,
            precision=lax.Precision.DEFAULT)
        over = flag[:, :1] > 0.5
        kbuf[slot] = jnp.where(over, kover, kbuf[slot])
        vbuf[slot] = jnp.where(over, vover, vbuf[slot])

    @pl.when(nb_cur > 0)
    def _():
        qr = _rope(q_ref[0], cosv, sinv)
        qtile = jnp.concatenate([qr] * KVH, axis=1)
        headsel = (lax.broadcasted_iota(jnp.int32, (H, ROWF), 1) // D) == (
            lax.broadcasted_iota(jnp.int32, (H, ROWF), 0) % KVH)
        qbig = jnp.where(headsel, qtile, 0.0).astype(jnp.bfloat16)
        kchunk = kbuf[slot].astype(jnp.bfloat16)
        vchunk = vbuf[slot].astype(jnp.bfloat16)
        s = lax.dot_general(
            qbig, kchunk, (((1,), (1,)), ((), ())),
            preferred_element_type=jnp.float32,
            precision=lax.Precision.DEFAULT) * SCALE
        pos = pos_ref[0]
        s = jnp.where(pos < nvalid, s, NEG)
        mprev = m_ref[:, :1]
        mcur = jnp.max(s, axis=1, keepdims=True)
        mnew = jnp.maximum(mprev, mcur)
        alpha = jnp.exp(mprev - mnew)
        p = jnp.exp(s - mnew)
        p = jnp.where(pos < nvalid, p, 0.0)
        lnew = l_ref[:, :1] * alpha + jnp.sum(p, axis=1, keepdims=True)
        pv = lax.dot_general(
            p.astype(jnp.bfloat16), vchunk, (((1,), (0,)), ((), ())),
            preferred_element_type=jnp.float32,
            precision=lax.Precision.DEFAULT)
        hsel = lax.broadcasted_iota(jnp.int32, (H, D), 0) % KVH
        pvh = jnp.zeros((H, D), jnp.float32)
        for h in range(KVH):
            pvh = jnp.where(hsel == h, pv[:, h * D:(h + 1) * D], pvh)
        acc_ref[...] = acc_ref[...] * alpha + pvh
        m_ref[...] = jnp.broadcast_to(mnew, (H, 128))
        l_ref[...] = jnp.broadcast_to(lnew, (H, 128))

    @pl.when(c == NCH - 1)
    def _():
        qr = _rope(q_ref[0], cosv, sinv)
        krow = kall_ref[pl.ds(b, 1)]
        vrow = vall_ref[pl.ds(b, 1)]
        krb = _rope(krow, cosv, sinv)
        kg = jnp.concatenate([krb[:, h, :] for h in range(KVH)], axis=0)
        vg = jnp.concatenate([vrow[:, h, :] for h in range(KVH)], axis=0)
        k_bc = jnp.concatenate([kg] * (H // KVH), axis=0)
        v_bc = jnp.concatenate([vg] * (H // KVH), axis=0)
        slast = jnp.sum(qr * k_bc, axis=1, keepdims=True) * SCALE
        mprev = m_ref[:, :1]
        mnew = jnp.maximum(mprev, slast)
        alpha = jnp.exp(mprev - mnew)
        pl_ = jnp.exp(slast - mnew)
        lfin = l_ref[:, :1] * alpha + pl_
        out_ref[0] = (acc_ref[...] * alpha + pl_ * v_bc) / lfin


@jax.jit
def kernel(Q, K, V, Kcache, Vcache, input_length, save_slots, fetch_slots,
           cos, sin):
    kc2 = Kcache.reshape(CACHE, ROWF)
    vc2 = Vcache.reshape(CACHE, ROWF)
    start8 = (fetch_slots // 8) * 8
    off = fetch_slots - start8
    w = jnp.arange(RB, dtype=jnp.int32)
    pib = w[None, None, :] - off[:, :, None]
    inblk = (pib >= 0) & (pib < BLOCK)
    n = jnp.arange(NSLOTS, dtype=jnp.int32)
    posf = n[None, :, None] * BLOCK + pib
    posf = jnp.where(inblk, posf, BIGPOS)
    idx = (start8[:, :, None] + w[None, None, :]).reshape(B * NCH, 1, CR)
    posf = posf.reshape(B * NCH, 1, CR)
    cs = jnp.stack([cos[0], sin[0]]).reshape(1, 2)

    ss = save_slots.reshape(B)
    inwin = (ss[None, None, :] >= start8[:, :, None]) & (
        ss[None, None, :] < start8[:, :, None] + RB)
    earlier = jnp.arange(B)[None, :] < jnp.arange(B)[:, None]
    hitblk = jnp.any(inwin & earlier[:, None, :], axis=2)
    nvalid_h = jnp.maximum(input_length - 1, 0)
    nblk_h = (nvalid_h + BLOCK - 1) // BLOCK
    needed = jnp.arange(NSLOTS)[None, :] < nblk_h[:, None]
    hitflags = jnp.any((hitblk & needed).reshape(B, NCH, BPC),
                       axis=2).reshape(T).astype(jnp.int32)

    grid_spec = pltpu.PrefetchScalarGridSpec(
        num_scalar_prefetch=4,
        grid=(T,),
        in_specs=[
            pl.BlockSpec((1, H, D), lambda t, *_: (t // NCH, 0, 0)),
            pl.BlockSpec((B, KVH, D), lambda t, *_: (0, 0, 0)),
            pl.BlockSpec((B, KVH, D), lambda t, *_: (0, 0, 0)),
            pl.BlockSpec((1, 1, CR), lambda t, *_: (t, 0, 0)),
            pl.BlockSpec((1, 1, CR), lambda t, *_: (t, 0, 0)),
            pl.BlockSpec((1, 2), lambda t, *_: (0, 0)),
            pl.BlockSpec(memory_space=pltpu.MemorySpace.HBM),
            pl.BlockSpec(memory_space=pltpu.MemorySpace.HBM),
        ],
        out_specs=pl.BlockSpec((1, H, D), lambda t, *_: (t // NCH, 0, 0)),
        scratch_shapes=[
            pltpu.VMEM((NBUF, CR, ROWF), jnp.float32),
            pltpu.VMEM((NBUF, CR, ROWF), jnp.float32),
            pltpu.VMEM((H, D), jnp.float32),
            pltpu.VMEM((H, 128), jnp.float32),
            pltpu.VMEM((H, 128), jnp.float32),
            pltpu.SemaphoreType.DMA((NBUF,)),
            pltpu.SemaphoreType.DMA((NBUF,)),
        ],
    )
    return pl.pallas_call(
        _body,
        grid_spec=grid_spec,
        out_shape=jax.ShapeDtypeStruct((B, H, D), jnp.float32),
        compiler_params=pltpu.CompilerParams(
            dimension_semantics=("arbitrary",)),
    )(fetch_slots, input_length, save_slots.reshape(B), hitflags, Q, K, V,
      idx, posf, cs, kc2, vc2)


# speedup vs baseline: 2.9664x; 2.9664x over previous
import functools

import jax
import jax.numpy as jnp
from jax import lax
from jax.experimental import pallas as pl
from jax.experimental.pallas import tpu as pltpu

B = 32
H = 32
KVH = 4
D = 128
CACHE = 131072
BLOCK = 16
NSLOTS = 256
L_GATHER = NSLOTS * BLOCK

RB = 16
BPC = 64
CR = BPC * RB
NCH = NSLOTS // BPC
T = B * NCH
BIGPOS = 2 ** 30
NBUF = 3

ROWF = KVH * D
SCALE = 1.0 / (D ** 0.5)
NEG = -1e30


def _rope(x, cosv, sinv):
    half = jnp.concatenate([x[..., 64:], x[..., :64]], axis=-1)
    coef = jnp.where(
        lax.broadcasted_iota(jnp.int32, x.shape, x.ndim - 1) < 64, -1.0, 1.0)
    return x * cosv + half * coef * sinv


GRP = 8


def _body(fs_ref, il_ref, ss_ref, hit_ref,
          q_ref, kall_ref, vall_ref, idx_ref, cs_ref, kc3_ref,
          vc3_ref,
          out_ref,
          kbuf, vbuf, acc_ref, m_ref, l_ref, ksem, vsem):
    kc_ref = kc3_ref.reshape(CACHE, ROWF)
    vc_ref = vc3_ref.reshape(CACHE, ROWF)
    t = pl.program_id(0)
    b = t // NCH
    c = t % NCH

    def nblocks(bb):
        nv = jnp.maximum(il_ref[bb] - 1, 0)
        return (nv + BLOCK - 1) // BLOCK

    def issue(tt, sl):
        bb = tt // NCH
        cc = tt % NCH
        nb = jnp.clip(nblocks(bb) - cc * BPC, 0, BPC)
        for g in range(BPC // GRP):
            @pl.when(nb > g * GRP)
            def _():
                for j in range(GRP):
                    i = g * GRP + j
                    i_eff = jnp.minimum(i, nb - 1)
                    row = fs_ref[bb, cc * BPC + i_eff]
                    pltpu.make_async_copy(
                        kc_ref.at[pl.ds(row, RB)],
                        kbuf.at[sl, pl.ds(i * RB, RB)],
                        ksem.at[sl]).start()
                    pltpu.make_async_copy(
                        vc_ref.at[pl.ds(row, RB)],
                        vbuf.at[sl, pl.ds(i * RB, RB)],
                        vsem.at[sl]).start()

    @pl.when(t == 0)
    def _():
        kbuf[...] = jnp.zeros((NBUF, CR, ROWF), jnp.float32)
        vbuf[...] = jnp.zeros((NBUF, CR, ROWF), jnp.float32)
        issue(0, 0)
        issue(1, 1)

    @pl.when(t + 2 < T)
    def _():
        issue(t + 2, (t + 2) % NBUF)

    slot = t % NBUF
    nb_cur = jnp.clip(nblocks(b) - c * BPC, 0, BPC)
    nb_iss = ((nb_cur + GRP - 1) // GRP) * GRP

    @pl.when(nb_iss == BPC)
    def _():
        pltpu.make_async_copy(
            kc_ref.at[pl.ds(0, CR)], kbuf.at[slot], ksem.at[slot]).wait()
        pltpu.make_async_copy(
            vc_ref.at[pl.ds(0, CR)], vbuf.at[slot], vsem.at[slot]).wait()

    @pl.when(jnp.logical_and(nb_iss > 0, nb_iss < BPC))
    def _():
        def wait_one(i, _):
            pltpu.make_async_copy(
                kc_ref.at[pl.ds(0, GRP * RB)],
                kbuf.at[slot, pl.ds(i * GRP * RB, GRP * RB)],
                ksem.at[slot]).wait()
            pltpu.make_async_copy(
                vc_ref.at[pl.ds(0, GRP * RB)],
                vbuf.at[slot, pl.ds(i * GRP * RB, GRP * RB)],
                vsem.at[slot]).wait()
            return 0

        lax.fori_loop(0, nb_iss // GRP, wait_one, 0)

    @pl.when(c == 0)
    def _():
        m_ref[...] = jnp.full((H, 128), NEG, jnp.float32)
        l_ref[...] = jnp.zeros((H, 128), jnp.float32)
        acc_ref[...] = jnp.zeros((H, D), jnp.float32)

    cosv = cs_ref[0, 0]
    sinv = cs_ref[0, 1]

    nvalid = jnp.maximum(il_ref[b] - 1, 0)

    @pl.when(hit_ref[t] > 0)
    def _():
        ka = kall_ref[...]
        kr3 = _rope(ka, cosv, sinv)
        kr_flat = jnp.concatenate(
            [kr3[:, h, :] for h in range(KVH)], axis=1)
        v_flat = jnp.concatenate(
            [vall_ref[:, h, :] for h in range(KVH)], axis=1)
        idxblk2 = idx_ref[0]
        wr = jnp.full((1, CR), -1, jnp.int32)
        for bp in range(B):
            hit = jnp.logical_and(idxblk2 == ss_ref[bp], bp < b)
            wr = jnp.where(hit, bp, wr)
        onehot = (lax.broadcasted_iota(jnp.int32, (B, CR), 0) == wr
                  ).astype(jnp.float32)
        kover = lax.dot_general(
            onehot, kr_flat, (((0,), (0,)), ((), ())),
            preferred_element_type=jnp.float32,
            precision=lax.Precision.DEFAULT)
        vover = lax.dot_general(
            onehot, v_flat, (((0,), (0,)), ((), ())),
            preferred_element_type=jnp.float32,
            precision=lax.Precision.DEFAULT)
        flag = lax.dot_general(
            onehot, jnp.ones((B, 128), jnp.float32),
            (((0,), (0,)), ((), ())),
            preferred_element_type=jnp.float32,
            precision=lax.Precision.DEFAULT)
        over = flag[:, :1] > 0.5
        kbuf[slot] = jnp.where(over, kover, kbuf[slot])
        vbuf[slot] = jnp.where(over, vover, vbuf[slot])

    @pl.when(nb_cur > 0)
    def _():
        qr = _rope(q_ref[0], cosv, sinv)
        qtile = jnp.concatenate([qr] * KVH, axis=1)
        headsel = (lax.broadcasted_iota(jnp.int32, (H, ROWF), 1) // D) == (
            lax.broadcasted_iota(jnp.int32, (H, ROWF), 0) % KVH)
        qbig = jnp.where(headsel, qtile, 0.0).astype(jnp.bfloat16)
        kchunk = kbuf[slot].astype(jnp.bfloat16)
        vchunk = vbuf[slot].astype(jnp.bfloat16)
        s = lax.dot_general(
            qbig, kchunk, (((1,), (1,)), ((), ())),
            preferred_element_type=jnp.float32,
            precision=lax.Precision.DEFAULT) * SCALE
        pos = c * CR + lax.broadcasted_iota(jnp.int32, (1, CR), 1)
        s = jnp.where(pos < nvalid, s, NEG)
        mprev = m_ref[:, :1]
        mcur = jnp.max(s, axis=1, keepdims=True)
        mnew = jnp.maximum(mprev, mcur)
        alpha = jnp.exp(mprev - mnew)
        p = jnp.exp(s - mnew)
        p = jnp.where(pos < nvalid, p, 0.0)
        lnew = l_ref[:, :1] * alpha + jnp.sum(p, axis=1, keepdims=True)
        pv = lax.dot_general(
            p.astype(jnp.bfloat16), vchunk, (((1,), (0,)), ((), ())),
            preferred_element_type=jnp.float32,
            precision=lax.Precision.DEFAULT)
        hsel = lax.broadcasted_iota(jnp.int32, (H, D), 0) % KVH
        pvh = jnp.zeros((H, D), jnp.float32)
        for h in range(KVH):
            pvh = jnp.where(hsel == h, pv[:, h * D:(h + 1) * D], pvh)
        acc_ref[...] = acc_ref[...] * alpha + pvh
        m_ref[...] = jnp.broadcast_to(mnew, (H, 128))
        l_ref[...] = jnp.broadcast_to(lnew, (H, 128))

    @pl.when(c == NCH - 1)
    def _():
        qr = _rope(q_ref[0], cosv, sinv)
        krow = kall_ref[pl.ds(b, 1)]
        vrow = vall_ref[pl.ds(b, 1)]
        krb = _rope(krow, cosv, sinv)
        kg = jnp.concatenate([krb[:, h, :] for h in range(KVH)], axis=0)
        vg = jnp.concatenate([vrow[:, h, :] for h in range(KVH)], axis=0)
        k_bc = jnp.concatenate([kg] * (H // KVH), axis=0)
        v_bc = jnp.concatenate([vg] * (H // KVH), axis=0)
        slast = jnp.sum(qr * k_bc, axis=1, keepdims=True) * SCALE
        mprev = m_ref[:, :1]
        mnew = jnp.maximum(mprev, slast)
        alpha = jnp.exp(mprev - mnew)
        pl_ = jnp.exp(slast - mnew)
        lfin = l_ref[:, :1] * alpha + pl_
        out_ref[0] = (acc_ref[...] * alpha + pl_ * v_bc) / lfin


@jax.jit
def kernel(Q, K, V, Kcache, Vcache, input_length, save_slots, fetch_slots,
           cos, sin):
    w = jnp.arange(RB, dtype=jnp.int32)
    idx = (fetch_slots[:, :, None] + w[None, None, :]).reshape(
        B * NCH, 1, CR)
    cs = jnp.stack([cos[0], sin[0]]).reshape(1, 2)

    ss = save_slots.reshape(B)
    inwin = (ss[None, None, :] >= fetch_slots[:, :, None]) & (
        ss[None, None, :] < fetch_slots[:, :, None] + RB)
    earlier = jnp.arange(B)[None, :] < jnp.arange(B)[:, None]
    hitblk = jnp.any(inwin & earlier[:, None, :], axis=2)
    nvalid_h = jnp.maximum(input_length - 1, 0)
    nblk_h = (nvalid_h + BLOCK - 1) // BLOCK
    needed = jnp.arange(NSLOTS)[None, :] < nblk_h[:, None]
    hitflags = jnp.any((hitblk & needed).reshape(B, NCH, BPC),
                       axis=2).reshape(T).astype(jnp.int32)

    grid_spec = pltpu.PrefetchScalarGridSpec(
        num_scalar_prefetch=4,
        grid=(T,),
        in_specs=[
            pl.BlockSpec((1, H, D), lambda t, *_: (t // NCH, 0, 0)),
            pl.BlockSpec((B, KVH, D), lambda t, *_: (0, 0, 0)),
            pl.BlockSpec((B, KVH, D), lambda t, *_: (0, 0, 0)),
            pl.BlockSpec((1, 1, CR), lambda t, *_: (t, 0, 0)),
            pl.BlockSpec((1, 2), lambda t, *_: (0, 0)),
            pl.BlockSpec(memory_space=pltpu.MemorySpace.HBM),
            pl.BlockSpec(memory_space=pltpu.MemorySpace.HBM),
        ],
        out_specs=pl.BlockSpec((1, H, D), lambda t, *_: (t // NCH, 0, 0)),
        scratch_shapes=[
            pltpu.VMEM((NBUF, CR, ROWF), jnp.float32),
            pltpu.VMEM((NBUF, CR, ROWF), jnp.float32),
            pltpu.VMEM((H, D), jnp.float32),
            pltpu.VMEM((H, 128), jnp.float32),
            pltpu.VMEM((H, 128), jnp.float32),
            pltpu.SemaphoreType.DMA((NBUF,)),
            pltpu.SemaphoreType.DMA((NBUF,)),
        ],
    )
    return pl.pallas_call(
        _body,
        grid_spec=grid_spec,
        out_shape=jax.ShapeDtypeStruct((B, H, D), jnp.float32),
        compiler_params=pltpu.CompilerParams(
            dimension_semantics=("arbitrary",)),
    )(fetch_slots, input_length, save_slots.reshape(B), hitflags, Q, K, V,
      idx, cs, Kcache, Vcache)


# speedup vs baseline: 3.2352x; 1.0906x over previous
import functools

import jax
import jax.numpy as jnp
from jax import lax
from jax.experimental import pallas as pl
from jax.experimental.pallas import tpu as pltpu

B = 32
H = 32
KVH = 4
D = 128
CACHE = 131072
BLOCK = 16
NSLOTS = 256
L_GATHER = NSLOTS * BLOCK

RB = 16
BPC = 128
CR = BPC * RB
NCH = NSLOTS // BPC
T = B * NCH
BIGPOS = 2 ** 30
NBUF = 3

ROWF = KVH * D
SCALE = 1.0 / (D ** 0.5)
NEG = -1e30


def _rope(x, cosv, sinv):
    half = jnp.concatenate([x[..., 64:], x[..., :64]], axis=-1)
    coef = jnp.where(
        lax.broadcasted_iota(jnp.int32, x.shape, x.ndim - 1) < 64, -1.0, 1.0)
    return x * cosv + half * coef * sinv


GRP = 8


def _body(fs_ref, il_ref, ss_ref, hit_ref,
          q_ref, kall_ref, vall_ref, idx_ref, cs_ref, kc3_ref,
          vc3_ref,
          out_ref,
          kbuf, vbuf, acc_ref, m_ref, l_ref, ksem, vsem):
    kc_ref = kc3_ref.reshape(CACHE, ROWF)
    vc_ref = vc3_ref.reshape(CACHE, ROWF)
    t = pl.program_id(0)
    b = t // NCH
    c = t % NCH

    def nblocks(bb):
        nv = jnp.maximum(il_ref[bb] - 1, 0)
        return (nv + BLOCK - 1) // BLOCK

    def issue(tt, sl):
        bb = tt // NCH
        cc = tt % NCH
        nb = jnp.clip(nblocks(bb) - cc * BPC, 0, BPC)
        for g in range(BPC // GRP):
            @pl.when(nb > g * GRP)
            def _():
                for j in range(GRP):
                    i = g * GRP + j
                    i_eff = jnp.minimum(i, nb - 1)
                    row = fs_ref[bb, cc * BPC + i_eff]
                    pltpu.make_async_copy(
                        kc_ref.at[pl.ds(row, RB)],
                        kbuf.at[sl, pl.ds(i * RB, RB)],
                        ksem.at[sl]).start()
                    pltpu.make_async_copy(
                        vc_ref.at[pl.ds(row, RB)],
                        vbuf.at[sl, pl.ds(i * RB, RB)],
                        vsem.at[sl]).start()

    @pl.when(t == 0)
    def _():
        kbuf[...] = jnp.zeros((NBUF, CR, ROWF), jnp.float32)
        vbuf[...] = jnp.zeros((NBUF, CR, ROWF), jnp.float32)
        issue(0, 0)
        issue(1, 1)

    @pl.when(t + 2 < T)
    def _():
        issue(t + 2, (t + 2) % NBUF)

    slot = t % NBUF
    nb_cur = jnp.clip(nblocks(b) - c * BPC, 0, BPC)
    nb_iss = ((nb_cur + GRP - 1) // GRP) * GRP

    @pl.when(nb_iss == BPC)
    def _():
        pltpu.make_async_copy(
            kc_ref.at[pl.ds(0, CR)], kbuf.at[slot], ksem.at[slot]).wait()
        pltpu.make_async_copy(
            vc_ref.at[pl.ds(0, CR)], vbuf.at[slot], vsem.at[slot]).wait()

    @pl.when(jnp.logical_and(nb_iss > 0, nb_iss < BPC))
    def _():
        def wait_one(i, _):
            pltpu.make_async_copy(
                kc_ref.at[pl.ds(0, GRP * RB)],
                kbuf.at[slot, pl.ds(i * GRP * RB, GRP * RB)],
                ksem.at[slot]).wait()
            pltpu.make_async_copy(
                vc_ref.at[pl.ds(0, GRP * RB)],
                vbuf.at[slot, pl.ds(i * GRP * RB, GRP * RB)],
                vsem.at[slot]).wait()
            return 0

        lax.fori_loop(0, nb_iss // GRP, wait_one, 0)

    @pl.when(c == 0)
    def _():
        m_ref[...] = jnp.full((H, 128), NEG, jnp.float32)
        l_ref[...] = jnp.zeros((H, 128), jnp.float32)
        acc_ref[...] = jnp.zeros((H, D), jnp.float32)

    cosv = cs_ref[0, 0]
    sinv = cs_ref[0, 1]

    nvalid = jnp.maximum(il_ref[b] - 1, 0)

    @pl.when(hit_ref[t] > 0)
    def _():
        ka = kall_ref[...]
        kr3 = _rope(ka, cosv, sinv)
        kr_flat = jnp.concatenate(
            [kr3[:, h, :] for h in range(KVH)], axis=1)
        v_flat = jnp.concatenate(
            [vall_ref[:, h, :] for h in range(KVH)], axis=1)
        idxblk2 = idx_ref[0]
        wr = jnp.full((1, CR), -1, jnp.int32)
        for bp in range(B):
            hit = jnp.logical_and(idxblk2 == ss_ref[bp], bp < b)
            wr = jnp.where(hit, bp, wr)
        onehot = (lax.broadcasted_iota(jnp.int32, (B, CR), 0) == wr
                  ).astype(jnp.float32)
        kover = lax.dot_general(
            onehot, kr_flat, (((0,), (0,)), ((), ())),
            preferred_element_type=jnp.float32,
            precision=lax.Precision.DEFAULT)
        vover = lax.dot_general(
            onehot, v_flat, (((0,), (0,)), ((), ())),
            preferred_element_type=jnp.float32,
            precision=lax.Precision.DEFAULT)
        flag = lax.dot_general(
            onehot, jnp.ones((B, 128), jnp.float32),
            (((0,), (0,)), ((), ())),
            preferred_element_type=jnp.float32,
            precision=lax.Precision.DEFAULT)
        over = flag[:, :1] > 0.5
        kbuf[slot] = jnp.where(over, kover, kbuf[slot])
        vbuf[slot] = jnp.where(over, vover, vbuf[slot])

    @pl.when(nb_cur > 0)
    def _():
        qr = _rope(q_ref[0], cosv, sinv)
        qtile = jnp.concatenate([qr] * KVH, axis=1)
        headsel = (lax.broadcasted_iota(jnp.int32, (H, ROWF), 1) // D) == (
            lax.broadcasted_iota(jnp.int32, (H, ROWF), 0) % KVH)
        qbig = jnp.where(headsel, qtile, 0.0).astype(jnp.bfloat16)
        kchunk = kbuf[slot].astype(jnp.bfloat16)
        vchunk = vbuf[slot].astype(jnp.bfloat16)
        s = lax.dot_general(
            qbig, kchunk, (((1,), (1,)), ((), ())),
            preferred_element_type=jnp.float32,
            precision=lax.Precision.DEFAULT) * SCALE
        pos = c * CR + lax.broadcasted_iota(jnp.int32, (1, CR), 1)
        s = jnp.where(pos < nvalid, s, NEG)
        mprev = m_ref[:, :1]
        mcur = jnp.max(s, axis=1, keepdims=True)
        mnew = jnp.maximum(mprev, mcur)
        alpha = jnp.exp(mprev - mnew)
        p = jnp.exp(s - mnew)
        p = jnp.where(pos < nvalid, p, 0.0)
        lnew = l_ref[:, :1] * alpha + jnp.sum(p, axis=1, keepdims=True)
        pv = lax.dot_general(
            p.astype(jnp.bfloat16), vchunk, (((1,), (0,)), ((), ())),
            preferred_element_type=jnp.float32,
            precision=lax.Precision.DEFAULT)
        hsel = lax.broadcasted_iota(jnp.int32, (H, D), 0) % KVH
        pvh = jnp.zeros((H, D), jnp.float32)
        for h in range(KVH):
            pvh = jnp.where(hsel == h, pv[:, h * D:(h + 1) * D], pvh)
        acc_ref[...] = acc_ref[...] * alpha + pvh
        m_ref[...] = jnp.broadcast_to(mnew, (H, 128))
        l_ref[...] = jnp.broadcast_to(lnew, (H, 128))

    @pl.when(c == NCH - 1)
    def _():
        qr = _rope(q_ref[0], cosv, sinv)
        krow = kall_ref[pl.ds(b, 1)]
        vrow = vall_ref[pl.ds(b, 1)]
        krb = _rope(krow, cosv, sinv)
        kg = jnp.concatenate([krb[:, h, :] for h in range(KVH)], axis=0)
        vg = jnp.concatenate([vrow[:, h, :] for h in range(KVH)], axis=0)
        k_bc = jnp.concatenate([kg] * (H // KVH), axis=0)
        v_bc = jnp.concatenate([vg] * (H // KVH), axis=0)
        slast = jnp.sum(qr * k_bc, axis=1, keepdims=True) * SCALE
        mprev = m_ref[:, :1]
        mnew = jnp.maximum(mprev, slast)
        alpha = jnp.exp(mprev - mnew)
        pl_ = jnp.exp(slast - mnew)
        lfin = l_ref[:, :1] * alpha + pl_
        out_ref[0] = (acc_ref[...] * alpha + pl_ * v_bc) / lfin


@jax.jit
def kernel(Q, K, V, Kcache, Vcache, input_length, save_slots, fetch_slots,
           cos, sin):
    w = jnp.arange(RB, dtype=jnp.int32)
    idx = (fetch_slots[:, :, None] + w[None, None, :]).reshape(
        B * NCH, 1, CR)
    cs = jnp.stack([cos[0], sin[0]]).reshape(1, 2)

    ss = save_slots.reshape(B)
    inwin = (ss[None, None, :] >= fetch_slots[:, :, None]) & (
        ss[None, None, :] < fetch_slots[:, :, None] + RB)
    earlier = jnp.arange(B)[None, :] < jnp.arange(B)[:, None]
    hitblk = jnp.any(inwin & earlier[:, None, :], axis=2)
    nvalid_h = jnp.maximum(input_length - 1, 0)
    nblk_h = (nvalid_h + BLOCK - 1) // BLOCK
    needed = jnp.arange(NSLOTS)[None, :] < nblk_h[:, None]
    hitflags = jnp.any((hitblk & needed).reshape(B, NCH, BPC),
                       axis=2).reshape(T).astype(jnp.int32)

    grid_spec = pltpu.PrefetchScalarGridSpec(
        num_scalar_prefetch=4,
        grid=(T,),
        in_specs=[
            pl.BlockSpec((1, H, D), lambda t, *_: (t // NCH, 0, 0)),
            pl.BlockSpec((B, KVH, D), lambda t, *_: (0, 0, 0)),
            pl.BlockSpec((B, KVH, D), lambda t, *_: (0, 0, 0)),
            pl.BlockSpec((1, 1, CR), lambda t, *_: (t, 0, 0)),
            pl.BlockSpec((1, 2), lambda t, *_: (0, 0)),
            pl.BlockSpec(memory_space=pltpu.MemorySpace.HBM),
            pl.BlockSpec(memory_space=pltpu.MemorySpace.HBM),
        ],
        out_specs=pl.BlockSpec((1, H, D), lambda t, *_: (t // NCH, 0, 0)),
        scratch_shapes=[
            pltpu.VMEM((NBUF, CR, ROWF), jnp.float32),
            pltpu.VMEM((NBUF, CR, ROWF), jnp.float32),
            pltpu.VMEM((H, D), jnp.float32),
            pltpu.VMEM((H, 128), jnp.float32),
            pltpu.VMEM((H, 128), jnp.float32),
            pltpu.SemaphoreType.DMA((NBUF,)),
            pltpu.SemaphoreType.DMA((NBUF,)),
        ],
    )
    return pl.pallas_call(
        _body,
        grid_spec=grid_spec,
        out_shape=jax.ShapeDtypeStruct((B, H, D), jnp.float32),
        compiler_params=pltpu.CompilerParams(
            dimension_semantics=("arbitrary",)),
    )(fetch_slots, input_length, save_slots.reshape(B), hitflags, Q, K, V,
      idx, cs, Kcache, Vcache)
